# baseline (speedup 1.0000x reference)
import functools

import jax
import jax.numpy as jnp
from jax import lax
from jax.experimental import pallas as pl
from jax.experimental.pallas import tpu as pltpu
from jax.experimental.pallas import tpu_sc as plsc

NC = 2
NS = 16
NW = NC * NS
L = 16
CHUNK = 128
NBUF = 2
SB = 16
G = 64
BN = 1000


def _mesh():
    return plsc.VectorSubcoreMesh(
        core_axis_name="c", subcore_axis_name="s",
        num_cores=NC, num_subcores=NS)


def _sc_degree(dst2, npad):
    tpt = dst2.shape[1]

    @functools.partial(
        pl.kernel,
        out_type=jax.ShapeDtypeStruct((NW, npad), jnp.float32),
        mesh=_mesh(),
        compiler_params=pltpu.CompilerParams(needs_layout_passes=False),
        scratch_types=[
            pltpu.VMEM((tpt,), jnp.int32),
            pltpu.VMEM((npad,), jnp.float32),
        ],
    )
    def k(dst_hbm, out_hbm, dstv, hist):
        c = lax.axis_index("c")
        s = lax.axis_index("s")
        wid = s * NC + c
        zero16 = jnp.zeros((L,), jnp.float32)

        def zbody(i, carry):
            hist[pl.ds(i * L, L)] = zero16
            return carry

        lax.fori_loop(0, npad // L, zbody, 0)
        pltpu.sync_copy(dst_hbm.at[wid], dstv)
        one16 = jnp.ones((L,), jnp.float32)

        def body(i, carry):
            idx = dstv[pl.ds(i * L, L)]
            plsc.addupdate_scatter(hist, [idx], one16)
            return carry

        lax.fori_loop(0, tpt // L, body, 0)
        pltpu.sync_copy(hist, out_hbm.at[wid])

    return k(dst2)


def _sc_propagate(xws, src3, dst3, zrows, npad):
    kchunks = src3.shape[1]
    nsb = kchunks // SB
    h = xws.shape[1]
    rpt = npad // NS

    @functools.partial(
        pl.kernel,
        out_type=jax.ShapeDtypeStruct((NC, npad, h), jnp.float32),
        mesh=_mesh(),
        compiler_params=pltpu.CompilerParams(needs_layout_passes=False),
        scratch_types=[
            pltpu.VMEM((2, SB, CHUNK), jnp.int32),
            pltpu.VMEM((2, SB, CHUNK), jnp.int32),
            pltpu.VMEM((NBUF, CHUNK, h), jnp.float32),
            pltpu.VMEM_SHARED((npad, h), jnp.float32),
            pltpu.SemaphoreType.DMA,
            pltpu.SemaphoreType.DMA,
            pltpu.SemaphoreType.DMA,
        ],
    )
    def k(xws_hbm, src_hbm, dst_hbm, z_hbm, out_hbm, srcv, dstv, rows, acc,
          gsem, ssem, isem):
        c = lax.axis_index("c")
        s = lax.axis_index("s")
        wid = s * NC + c
        base = s * rpt
        pltpu.sync_copy(z_hbm.at[pl.ds(base, rpt)], acc.at[pl.ds(base, rpt)])
        pltpu.sync_copy(src_hbm.at[wid, pl.ds(0, SB)], srcv.at[0])
        pltpu.sync_copy(dst_hbm.at[wid, pl.ds(0, SB)], dstv.at[0])
        plsc.subcore_barrier()

        def outer(sb, carry):
            slot = sb % 2

            @pl.when(sb + 1 < nsb)
            def _():
                nxt = (sb + 1) % 2
                off = (sb + 1) * SB
                pltpu.async_copy(
                    src_hbm.at[wid, pl.ds(off, SB)], srcv.at[nxt], isem)
                pltpu.async_copy(
                    dst_hbm.at[wid, pl.ds(off, SB)], dstv.at[nxt], isem)

            pltpu.async_copy(xws_hbm.at[srcv.at[slot, 0]], rows.at[0], gsem)

            def inner(t, carry2):
                pltpu.make_async_copy(
                    xws_hbm.at[srcv.at[slot, t]], rows.at[t % NBUF],
                    gsem).wait()

                @pl.when(t >= 1)
                def _():
                    pltpu.make_async_copy(
                        rows.at[(t - 1) % NBUF],
                        acc.at[dstv.at[slot, t - 1]], ssem).wait()

                @pl.when(t + 1 < SB)
                def _():
                    pltpu.async_copy(
                        xws_hbm.at[srcv.at[slot, t + 1]],
                        rows.at[(t + 1) % NBUF], gsem)

                pltpu.async_copy(
                    rows.at[t % NBUF], acc.at[dstv.at[slot, t]], ssem,
                    add=True)
                return carry2

            lax.fori_loop(0, SB, inner, 0)
            pltpu.make_async_copy(
                rows.at[(SB - 1) % NBUF], acc.at[dstv.at[slot, SB - 1]],
                ssem).wait()

            @pl.when(sb + 1 < nsb)
            def _():
                nxt = (sb + 1) % 2
                off = (sb + 1) * SB
                pltpu.make_async_copy(
                    src_hbm.at[wid, pl.ds(off, SB)], srcv.at[nxt],
                    isem).wait()
                pltpu.make_async_copy(
                    dst_hbm.at[wid, pl.ds(off, SB)], dstv.at[nxt],
                    isem).wait()

            return carry

        lax.fori_loop(0, nsb, outer, 0)
        plsc.subcore_barrier()
        pltpu.sync_copy(acc.at[pl.ds(base, rpt)], out_hbm.at[c, pl.ds(base, rpt)])

    return k(xws, src3, dst3, zrows)


def _tc_dinv(degp):
    nw, npad = degp.shape

    def body(dp_ref, o_ref):
        ones = jnp.ones((nw, 1), jnp.float32)
        deg = 1.0 + lax.dot_general(
            dp_ref[...], ones, (((0,), (0,)), ((), ())),
            preferred_element_type=jnp.float32,
        )
        o_ref[...] = lax.rsqrt(deg)

    return pl.pallas_call(
        body,
        out_shape=jax.ShapeDtypeStruct((npad, 1), jnp.float32),
    )(degp)


def _tc_first(x, w1, dinv2d):
    n, d = x.shape
    h = w1.shape[1]

    def body(x_ref, w_ref, dv_ref, o_ref):
        xw = jnp.dot(x_ref[...], w_ref[...], preferred_element_type=jnp.float32)
        o_ref[...] = xw * dv_ref[...]

    return pl.pallas_call(
        body,
        grid=(n // BN,),
        in_specs=[
            pl.BlockSpec((BN, d), lambda i: (i, 0)),
            pl.BlockSpec((d, h), lambda i: (0, 0)),
            pl.BlockSpec((BN, 1), lambda i: (i, 0)),
        ],
        out_specs=pl.BlockSpec((BN, h), lambda i: (i, 0)),
        out_shape=jax.ShapeDtypeStruct((n, h), jnp.float32),
    )(x, w1, dinv2d)


def _tc_layer(acc, xws, dinv2d, b2d, w):
    n, h = xws.shape
    h2 = w.shape[1]

    def body(a_ref, x_ref, dv_ref, b_ref, w_ref, o_ref):
        dinv = dv_ref[...]
        hpre = (a_ref[0] + a_ref[1] + x_ref[...]) * dinv + b_ref[...]
        hact = jnp.maximum(hpre, 0.0)
        o_ref[...] = (
            jnp.dot(hact, w_ref[...], preferred_element_type=jnp.float32)
            * dinv
        )

    return pl.pallas_call(
        body,
        grid=(n // BN,),
        in_specs=[
            pl.BlockSpec((NC, BN, h), lambda i: (0, i, 0)),
            pl.BlockSpec((BN, h), lambda i: (i, 0)),
            pl.BlockSpec((BN, 1), lambda i: (i, 0)),
            pl.BlockSpec((1, h), lambda i: (0, 0)),
            pl.BlockSpec((h, h2), lambda i: (0, 0)),
        ],
        out_specs=pl.BlockSpec((BN, h2), lambda i: (i, 0)),
        out_shape=jax.ShapeDtypeStruct((n, h2), jnp.float32),
    )(acc, xws, dinv2d, b2d, w)


def _tc_pool(acc, xws, dinv2d, b2d, batch2d):
    n, h = xws.shape

    def body(a_ref, x_ref, dv_ref, b_ref, bt_ref, sums_ref, cnt_ref):
        i = pl.program_id(0)
        hpre = (a_ref[0] + a_ref[1] + x_ref[...]) * dv_ref[...] + b_ref[...]
        hact = jnp.maximum(hpre, 0.0)
        onehot = (
            bt_ref[...] == lax.broadcasted_iota(jnp.int32, (1, G), 1)
        ).astype(jnp.float32)
        psums = lax.dot_general(
            onehot, hact, (((0,), (0,)), ((), ())),
            preferred_element_type=jnp.float32,
        )
        pcnts = lax.dot_general(
            onehot, jnp.ones((BN, 1), jnp.float32), (((0,), (0,)), ((), ())),
            preferred_element_type=jnp.float32,
        )

        @pl.when(i == 0)
        def _():
            sums_ref[...] = jnp.zeros_like(sums_ref)
            cnt_ref[...] = jnp.zeros_like(cnt_ref)

        sums_ref[...] += psums
        cnt_ref[...] += pcnts

    return pl.pallas_call(
        body,
        grid=(n // BN,),
        in_specs=[
            pl.BlockSpec((NC, BN, h), lambda i: (0, i, 0)),
            pl.BlockSpec((BN, h), lambda i: (i, 0)),
            pl.BlockSpec((BN, 1), lambda i: (i, 0)),
            pl.BlockSpec((1, h), lambda i: (0, 0)),
            pl.BlockSpec((BN, 1), lambda i: (i, 0)),
        ],
        out_specs=[
            pl.BlockSpec((G, h), lambda i: (0, 0)),
            pl.BlockSpec((G, 1), lambda i: (0, 0)),
        ],
        out_shape=[
            jax.ShapeDtypeStruct((G, h), jnp.float32),
            jax.ShapeDtypeStruct((G, 1), jnp.float32),
        ],
    )(acc, xws, dinv2d, b2d, batch2d)


def _tc_final(sums, cnts, wl, bl2d):
    def body(s_ref, c_ref, w_ref, b_ref, o_ref):
        pooled = s_ref[...] / jnp.maximum(c_ref[...], 1.0)
        o_ref[...] = (
            jnp.dot(pooled, w_ref[...], preferred_element_type=jnp.float32)
            + b_ref[...]
        )

    return pl.pallas_call(
        body,
        out_shape=jax.ShapeDtypeStruct((G, wl.shape[1]), jnp.float32),
    )(sums, cnts, wl, bl2d)


def kernel(x, edge_index, batch, W1, b1, W2, b2, W3, b3, Wl, bl):
    n = x.shape[0]
    e = edge_index.shape[1]
    h = W1.shape[1]

    npad = ((n + 1) + 127) // 128 * 128
    kchunks = (e + NW * CHUNK - 1) // (NW * CHUNK)
    kchunks = (kchunks + SB - 1) // SB * SB
    epad = NW * kchunks * CHUNK

    spare = npad - n
    pad_dst = n + (jnp.arange(epad - e, dtype=jnp.int32) % spare)
    pad_src = jnp.arange(epad - e, dtype=jnp.int32) % n
    src = jnp.concatenate([edge_index[0], pad_src])
    dst = jnp.concatenate([edge_index[1], pad_dst])
    src3 = src.reshape(NW, kchunks, CHUNK)
    dst3 = dst.reshape(NW, kchunks, CHUNK)
    dst2 = dst.reshape(NW, kchunks * CHUNK)
    zrows = jnp.zeros((npad, h), jnp.float32)
    b1r, b2r, b3r = b1.reshape(1, h), b2.reshape(1, h), b3.reshape(1, h)
    blr = bl.reshape(1, bl.shape[0])
    batch2d = batch.reshape(n, 1)

    degp = _sc_degree(dst2, npad)
    dinv2d = _tc_dinv(degp)
    xws1 = _tc_first(x, W1, dinv2d)
    acc1 = _sc_propagate(xws1, src3, dst3, zrows, npad)
    xws2 = _tc_layer(acc1, xws1, dinv2d, b1r, W2)
    acc2 = _sc_propagate(xws2, src3, dst3, zrows, npad)
    xws3 = _tc_layer(acc2, xws2, dinv2d, b2r, W3)
    acc3 = _sc_propagate(xws3, src3, dst3, zrows, npad)
    sums, cnts = _tc_pool(acc3, xws3, dinv2d, b3r, batch2d)
    return _tc_final(sums, cnts, Wl, blr)


# speedup vs baseline: 1.1469x; 1.1469x over previous
import functools

import jax
import jax.numpy as jnp
from jax import lax
from jax.experimental import pallas as pl
from jax.experimental.pallas import tpu as pltpu
from jax.experimental.pallas import tpu_sc as plsc

NC = 2
NS = 16
NW = NC * NS
L = 16
CHUNK = 128
NBUF = 2
SB = 16
G = 64
BN = 1000


def _mesh():
    return plsc.VectorSubcoreMesh(
        core_axis_name="c", subcore_axis_name="s",
        num_cores=NC, num_subcores=NS)


def _sc_degree(dst2, npad):
    tpt = dst2.shape[1]

    @functools.partial(
        pl.kernel,
        out_type=jax.ShapeDtypeStruct((NW, npad), jnp.float32),
        mesh=_mesh(),
        compiler_params=pltpu.CompilerParams(needs_layout_passes=False),
        scratch_types=[
            pltpu.VMEM((tpt,), jnp.int32),
            pltpu.VMEM((npad,), jnp.float32),
        ],
    )
    def k(dst_hbm, out_hbm, dstv, hist):
        c = lax.axis_index("c")
        s = lax.axis_index("s")
        wid = s * NC + c
        zero16 = jnp.zeros((L,), jnp.float32)

        def zbody(i, carry):
            hist[pl.ds(i * L, L)] = zero16
            return carry

        lax.fori_loop(0, npad // L, zbody, 0)
        pltpu.sync_copy(dst_hbm.at[wid], dstv)
        one16 = jnp.ones((L,), jnp.float32)

        def body(i, carry):
            idx = dstv[pl.ds(i * L, L)]
            plsc.addupdate_scatter(hist, [idx], one16)
            return carry

        lax.fori_loop(0, tpt // L, body, 0)
        pltpu.sync_copy(hist, out_hbm.at[wid])

    return k(dst2)


def _sc_propagate(xws, src3, dst3, zrows, npad):
    kchunks = src3.shape[1]
    nsb = kchunks // SB
    h = xws.shape[1]
    rpt = npad // NS

    @functools.partial(
        pl.kernel,
        out_type=jax.ShapeDtypeStruct((NC, npad, h), jnp.float32),
        mesh=_mesh(),
        compiler_params=pltpu.CompilerParams(needs_layout_passes=False),
        scratch_types=[
            pltpu.VMEM((2, SB, CHUNK), jnp.int32),
            pltpu.VMEM((2, SB, CHUNK), jnp.int32),
            pltpu.VMEM((NBUF, CHUNK, h), jnp.float32),
            pltpu.VMEM_SHARED((npad, h), jnp.float32),
            pltpu.SemaphoreType.DMA,
            pltpu.SemaphoreType.DMA,
            pltpu.SemaphoreType.DMA,
        ],
    )
    def k(xws_hbm, src_hbm, dst_hbm, z_hbm, out_hbm, srcv, dstv, rows, acc,
          gsem, ssem, isem):
        c = lax.axis_index("c")
        s = lax.axis_index("s")
        wid = s * NC + c
        base = s * rpt
        pltpu.sync_copy(z_hbm.at[pl.ds(base, rpt)], acc.at[pl.ds(base, rpt)])
        pltpu.sync_copy(src_hbm.at[wid, pl.ds(0, SB)], srcv.at[0])
        pltpu.sync_copy(dst_hbm.at[wid, pl.ds(0, SB)], dstv.at[0])
        plsc.subcore_barrier()

        def outer(sb, carry):
            slot = sb % 2

            @pl.when(sb + 1 < nsb)
            def _():
                nxt = (sb + 1) % 2
                off = (sb + 1) * SB
                pltpu.async_copy(
                    src_hbm.at[wid, pl.ds(off, SB)], srcv.at[nxt], isem)
                pltpu.async_copy(
                    dst_hbm.at[wid, pl.ds(off, SB)], dstv.at[nxt], isem)

            pltpu.async_copy(xws_hbm.at[srcv.at[slot, 0]], rows.at[0], gsem)
            pltpu.async_copy(xws_hbm.at[srcv.at[slot, 1]], rows.at[1], gsem)

            def inner(t, carry2):
                pltpu.make_async_copy(
                    xws_hbm.at[srcv.at[slot, t]], rows.at[t % NBUF],
                    gsem).wait()
                pltpu.sync_copy(
                    rows.at[t % NBUF], acc.at[dstv.at[slot, t]], add=True)

                @pl.when(t + 2 < SB)
                def _():
                    pltpu.async_copy(
                        xws_hbm.at[srcv.at[slot, t + 2]],
                        rows.at[(t + 2) % NBUF], gsem)
                return carry2

            lax.fori_loop(0, SB, inner, 0)

            @pl.when(sb + 1 < nsb)
            def _():
                nxt = (sb + 1) % 2
                off = (sb + 1) * SB
                pltpu.make_async_copy(
                    src_hbm.at[wid, pl.ds(off, SB)], srcv.at[nxt],
                    isem).wait()
                pltpu.make_async_copy(
                    dst_hbm.at[wid, pl.ds(off, SB)], dstv.at[nxt],
                    isem).wait()

            return carry

        lax.fori_loop(0, nsb, outer, 0)
        plsc.subcore_barrier()
        pltpu.sync_copy(acc.at[pl.ds(base, rpt)], out_hbm.at[c, pl.ds(base, rpt)])

    return k(xws, src3, dst3, zrows)


def _tc_dinv(degp):
    nw, npad = degp.shape

    def body(dp_ref, o_ref):
        ones = jnp.ones((nw, 1), jnp.float32)
        deg = 1.0 + lax.dot_general(
            dp_ref[...], ones, (((0,), (0,)), ((), ())),
            preferred_element_type=jnp.float32,
        )
        o_ref[...] = lax.rsqrt(deg)

    return pl.pallas_call(
        body,
        out_shape=jax.ShapeDtypeStruct((npad, 1), jnp.float32),
    )(degp)


def _tc_first(x, w1, dinv2d):
    n, d = x.shape
    h = w1.shape[1]

    def body(x_ref, w_ref, dv_ref, o_ref):
        xw = jnp.dot(x_ref[...], w_ref[...], preferred_element_type=jnp.float32)
        o_ref[...] = xw * dv_ref[...]

    return pl.pallas_call(
        body,
        grid=(n // BN,),
        in_specs=[
            pl.BlockSpec((BN, d), lambda i: (i, 0)),
            pl.BlockSpec((d, h), lambda i: (0, 0)),
            pl.BlockSpec((BN, 1), lambda i: (i, 0)),
        ],
        out_specs=pl.BlockSpec((BN, h), lambda i: (i, 0)),
        out_shape=jax.ShapeDtypeStruct((n, h), jnp.float32),
    )(x, w1, dinv2d)


def _tc_layer(acc, xws, dinv2d, b2d, w):
    n, h = xws.shape
    h2 = w.shape[1]

    def body(a_ref, x_ref, dv_ref, b_ref, w_ref, o_ref):
        dinv = dv_ref[...]
        hpre = (a_ref[0] + a_ref[1] + x_ref[...]) * dinv + b_ref[...]
        hact = jnp.maximum(hpre, 0.0)
        o_ref[...] = (
            jnp.dot(hact, w_ref[...], preferred_element_type=jnp.float32)
            * dinv
        )

    return pl.pallas_call(
        body,
        grid=(n // BN,),
        in_specs=[
            pl.BlockSpec((NC, BN, h), lambda i: (0, i, 0)),
            pl.BlockSpec((BN, h), lambda i: (i, 0)),
            pl.BlockSpec((BN, 1), lambda i: (i, 0)),
            pl.BlockSpec((1, h), lambda i: (0, 0)),
            pl.BlockSpec((h, h2), lambda i: (0, 0)),
        ],
        out_specs=pl.BlockSpec((BN, h2), lambda i: (i, 0)),
        out_shape=jax.ShapeDtypeStruct((n, h2), jnp.float32),
    )(acc, xws, dinv2d, b2d, w)


def _tc_pool(acc, xws, dinv2d, b2d, batch2d):
    n, h = xws.shape

    def body(a_ref, x_ref, dv_ref, b_ref, bt_ref, sums_ref, cnt_ref):
        i = pl.program_id(0)
        hpre = (a_ref[0] + a_ref[1] + x_ref[...]) * dv_ref[...] + b_ref[...]
        hact = jnp.maximum(hpre, 0.0)
        onehot = (
            bt_ref[...] == lax.broadcasted_iota(jnp.int32, (1, G), 1)
        ).astype(jnp.float32)
        psums = lax.dot_general(
            onehot, hact, (((0,), (0,)), ((), ())),
            preferred_element_type=jnp.float32,
        )
        pcnts = lax.dot_general(
            onehot, jnp.ones((BN, 1), jnp.float32), (((0,), (0,)), ((), ())),
            preferred_element_type=jnp.float32,
        )

        @pl.when(i == 0)
        def _():
            sums_ref[...] = jnp.zeros_like(sums_ref)
            cnt_ref[...] = jnp.zeros_like(cnt_ref)

        sums_ref[...] += psums
        cnt_ref[...] += pcnts

    return pl.pallas_call(
        body,
        grid=(n // BN,),
        in_specs=[
            pl.BlockSpec((NC, BN, h), lambda i: (0, i, 0)),
            pl.BlockSpec((BN, h), lambda i: (i, 0)),
            pl.BlockSpec((BN, 1), lambda i: (i, 0)),
            pl.BlockSpec((1, h), lambda i: (0, 0)),
            pl.BlockSpec((BN, 1), lambda i: (i, 0)),
        ],
        out_specs=[
            pl.BlockSpec((G, h), lambda i: (0, 0)),
            pl.BlockSpec((G, 1), lambda i: (0, 0)),
        ],
        out_shape=[
            jax.ShapeDtypeStruct((G, h), jnp.float32),
            jax.ShapeDtypeStruct((G, 1), jnp.float32),
        ],
    )(acc, xws, dinv2d, b2d, batch2d)


def _tc_final(sums, cnts, wl, bl2d):
    def body(s_ref, c_ref, w_ref, b_ref, o_ref):
        pooled = s_ref[...] / jnp.maximum(c_ref[...], 1.0)
        o_ref[...] = (
            jnp.dot(pooled, w_ref[...], preferred_element_type=jnp.float32)
            + b_ref[...]
        )

    return pl.pallas_call(
        body,
        out_shape=jax.ShapeDtypeStruct((G, wl.shape[1]), jnp.float32),
    )(sums, cnts, wl, bl2d)


def kernel(x, edge_index, batch, W1, b1, W2, b2, W3, b3, Wl, bl):
    n = x.shape[0]
    e = edge_index.shape[1]
    h = W1.shape[1]

    npad = ((n + 1) + 127) // 128 * 128
    kchunks = (e + NW * CHUNK - 1) // (NW * CHUNK)
    kchunks = (kchunks + SB - 1) // SB * SB
    epad = NW * kchunks * CHUNK

    spare = npad - n
    pad_dst = n + (jnp.arange(epad - e, dtype=jnp.int32) % spare)
    pad_src = jnp.arange(epad - e, dtype=jnp.int32) % n
    src = jnp.concatenate([edge_index[0], pad_src])
    dst = jnp.concatenate([edge_index[1], pad_dst])
    src3 = src.reshape(NW, kchunks, CHUNK)
    dst3 = dst.reshape(NW, kchunks, CHUNK)
    dst2 = dst.reshape(NW, kchunks * CHUNK)
    zrows = jnp.zeros((npad, h), jnp.float32)
    b1r, b2r, b3r = b1.reshape(1, h), b2.reshape(1, h), b3.reshape(1, h)
    blr = bl.reshape(1, bl.shape[0])
    batch2d = batch.reshape(n, 1)

    degp = _sc_degree(dst2, npad)
    dinv2d = _tc_dinv(degp)
    xws1 = _tc_first(x, W1, dinv2d)
    acc1 = _sc_propagate(xws1, src3, dst3, zrows, npad)
    xws2 = _tc_layer(acc1, xws1, dinv2d, b1r, W2)
    acc2 = _sc_propagate(xws2, src3, dst3, zrows, npad)
    xws3 = _tc_layer(acc2, xws2, dinv2d, b2r, W3)
    acc3 = _sc_propagate(xws3, src3, dst3, zrows, npad)
    sums, cnts = _tc_pool(acc3, xws3, dinv2d, b3r, batch2d)
    return _tc_final(sums, cnts, Wl, blr)


# speedup vs baseline: 1.1860x; 1.0340x over previous
import functools

import jax
import jax.numpy as jnp
from jax import lax
from jax.experimental import pallas as pl
from jax.experimental.pallas import tpu as pltpu
from jax.experimental.pallas import tpu_sc as plsc

NC = 2
NS = 16
NW = NC * NS
L = 16
CHUNK = 96
NBUF = 3
SB = 16
G = 64
BN = 1000


def _mesh():
    return plsc.VectorSubcoreMesh(
        core_axis_name="c", subcore_axis_name="s",
        num_cores=NC, num_subcores=NS)


def _sc_degree(dst2, npad):
    tpt = dst2.shape[1]

    @functools.partial(
        pl.kernel,
        out_type=jax.ShapeDtypeStruct((NW, npad), jnp.float32),
        mesh=_mesh(),
        compiler_params=pltpu.CompilerParams(needs_layout_passes=False),
        scratch_types=[
            pltpu.VMEM((tpt,), jnp.int32),
            pltpu.VMEM((npad,), jnp.float32),
        ],
    )
    def k(dst_hbm, out_hbm, dstv, hist):
        c = lax.axis_index("c")
        s = lax.axis_index("s")
        wid = s * NC + c
        zero16 = jnp.zeros((L,), jnp.float32)

        def zbody(i, carry):
            hist[pl.ds(i * L, L)] = zero16
            return carry

        lax.fori_loop(0, npad // L, zbody, 0)
        pltpu.sync_copy(dst_hbm.at[wid], dstv)
        one16 = jnp.ones((L,), jnp.float32)

        def body(i, carry):
            idx = dstv[pl.ds(i * L, L)]
            plsc.addupdate_scatter(hist, [idx], one16)
            return carry

        lax.fori_loop(0, tpt // L, body, 0)
        pltpu.sync_copy(hist, out_hbm.at[wid])

    return k(dst2)


def _sc_propagate(xws, src3, dst3, zrows, npad):
    kchunks = src3.shape[1]
    nsb = kchunks // SB
    h = xws.shape[1]
    rpt = npad // NS

    @functools.partial(
        pl.kernel,
        out_type=jax.ShapeDtypeStruct((NC, npad, h), jnp.float32),
        mesh=_mesh(),
        compiler_params=pltpu.CompilerParams(needs_layout_passes=False),
        scratch_types=[
            pltpu.VMEM((2, SB, CHUNK), jnp.int32),
            pltpu.VMEM((2, SB, CHUNK), jnp.int32),
            pltpu.VMEM((NBUF, CHUNK, h), jnp.float32),
            pltpu.VMEM_SHARED((npad, h), jnp.float32),
            pltpu.SemaphoreType.DMA,
            pltpu.SemaphoreType.DMA,
            pltpu.SemaphoreType.DMA,
        ],
    )
    def k(xws_hbm, src_hbm, dst_hbm, z_hbm, out_hbm, srcv, dstv, rows, acc,
          gsem, ssem, isem):
        c = lax.axis_index("c")
        s = lax.axis_index("s")
        wid = s * NC + c
        base = s * rpt
        pltpu.sync_copy(z_hbm.at[pl.ds(base, rpt)], acc.at[pl.ds(base, rpt)])
        pltpu.sync_copy(src_hbm.at[wid, pl.ds(0, SB)], srcv.at[0])
        pltpu.sync_copy(dst_hbm.at[wid, pl.ds(0, SB)], dstv.at[0])
        plsc.subcore_barrier()

        def outer(sb, carry):
            slot = sb % 2

            @pl.when(sb + 1 < nsb)
            def _():
                nxt = (sb + 1) % 2
                off = (sb + 1) * SB
                pltpu.async_copy(
                    src_hbm.at[wid, pl.ds(off, SB)], srcv.at[nxt], isem)
                pltpu.async_copy(
                    dst_hbm.at[wid, pl.ds(off, SB)], dstv.at[nxt], isem)

            pltpu.async_copy(xws_hbm.at[srcv.at[slot, 0]], rows.at[0], gsem)
            pltpu.async_copy(xws_hbm.at[srcv.at[slot, 1]], rows.at[1], gsem)

            def inner(t, carry2):
                pltpu.make_async_copy(
                    xws_hbm.at[srcv.at[slot, t]], rows.at[t % NBUF],
                    gsem).wait()
                pltpu.async_copy(
                    rows.at[t % NBUF], acc.at[dstv.at[slot, t]], ssem,
                    add=True)

                @pl.when(t >= 1)
                def _():
                    pltpu.make_async_copy(
                        rows.at[(t - 1) % NBUF],
                        acc.at[dstv.at[slot, t - 1]], ssem).wait()

                @pl.when(t + 2 < SB)
                def _():
                    pltpu.async_copy(
                        xws_hbm.at[srcv.at[slot, t + 2]],
                        rows.at[(t + 2) % NBUF], gsem)
                return carry2

            lax.fori_loop(0, SB, inner, 0)
            pltpu.make_async_copy(
                rows.at[(SB - 1) % NBUF], acc.at[dstv.at[slot, SB - 1]],
                ssem).wait()

            @pl.when(sb + 1 < nsb)
            def _():
                nxt = (sb + 1) % 2
                off = (sb + 1) * SB
                pltpu.make_async_copy(
                    src_hbm.at[wid, pl.ds(off, SB)], srcv.at[nxt],
                    isem).wait()
                pltpu.make_async_copy(
                    dst_hbm.at[wid, pl.ds(off, SB)], dstv.at[nxt],
                    isem).wait()

            return carry

        lax.fori_loop(0, nsb, outer, 0)
        plsc.subcore_barrier()
        pltpu.sync_copy(acc.at[pl.ds(base, rpt)], out_hbm.at[c, pl.ds(base, rpt)])

    return k(xws, src3, dst3, zrows)


def _tc_dinv(degp):
    nw, npad = degp.shape

    def body(dp_ref, o_ref):
        ones = jnp.ones((nw, 1), jnp.float32)
        deg = 1.0 + lax.dot_general(
            dp_ref[...], ones, (((0,), (0,)), ((), ())),
            preferred_element_type=jnp.float32,
        )
        o_ref[...] = lax.rsqrt(deg)

    return pl.pallas_call(
        body,
        out_shape=jax.ShapeDtypeStruct((npad, 1), jnp.float32),
    )(degp)


def _tc_first(x, w1, dinv2d):
    n, d = x.shape
    h = w1.shape[1]

    def body(x_ref, w_ref, dv_ref, o_ref):
        xw = jnp.dot(x_ref[...], w_ref[...], preferred_element_type=jnp.float32)
        o_ref[...] = xw * dv_ref[...]

    return pl.pallas_call(
        body,
        grid=(n // BN,),
        in_specs=[
            pl.BlockSpec((BN, d), lambda i: (i, 0)),
            pl.BlockSpec((d, h), lambda i: (0, 0)),
            pl.BlockSpec((BN, 1), lambda i: (i, 0)),
        ],
        out_specs=pl.BlockSpec((BN, h), lambda i: (i, 0)),
        out_shape=jax.ShapeDtypeStruct((n, h), jnp.float32),
    )(x, w1, dinv2d)


def _tc_layer(acc, xws, dinv2d, b2d, w):
    n, h = xws.shape
    h2 = w.shape[1]

    def body(a_ref, x_ref, dv_ref, b_ref, w_ref, o_ref):
        dinv = dv_ref[...]
        hpre = (a_ref[0] + a_ref[1] + x_ref[...]) * dinv + b_ref[...]
        hact = jnp.maximum(hpre, 0.0)
        o_ref[...] = (
            jnp.dot(hact, w_ref[...], preferred_element_type=jnp.float32)
            * dinv
        )

    return pl.pallas_call(
        body,
        grid=(n // BN,),
        in_specs=[
            pl.BlockSpec((NC, BN, h), lambda i: (0, i, 0)),
            pl.BlockSpec((BN, h), lambda i: (i, 0)),
            pl.BlockSpec((BN, 1), lambda i: (i, 0)),
            pl.BlockSpec((1, h), lambda i: (0, 0)),
            pl.BlockSpec((h, h2), lambda i: (0, 0)),
        ],
        out_specs=pl.BlockSpec((BN, h2), lambda i: (i, 0)),
        out_shape=jax.ShapeDtypeStruct((n, h2), jnp.float32),
    )(acc, xws, dinv2d, b2d, w)


def _tc_pool(acc, xws, dinv2d, b2d, batch2d):
    n, h = xws.shape

    def body(a_ref, x_ref, dv_ref, b_ref, bt_ref, sums_ref, cnt_ref):
        i = pl.program_id(0)
        hpre = (a_ref[0] + a_ref[1] + x_ref[...]) * dv_ref[...] + b_ref[...]
        hact = jnp.maximum(hpre, 0.0)
        onehot = (
            bt_ref[...] == lax.broadcasted_iota(jnp.int32, (1, G), 1)
        ).astype(jnp.float32)
        psums = lax.dot_general(
            onehot, hact, (((0,), (0,)), ((), ())),
            preferred_element_type=jnp.float32,
        )
        pcnts = lax.dot_general(
            onehot, jnp.ones((BN, 1), jnp.float32), (((0,), (0,)), ((), ())),
            preferred_element_type=jnp.float32,
        )

        @pl.when(i == 0)
        def _():
            sums_ref[...] = jnp.zeros_like(sums_ref)
            cnt_ref[...] = jnp.zeros_like(cnt_ref)

        sums_ref[...] += psums
        cnt_ref[...] += pcnts

    return pl.pallas_call(
        body,
        grid=(n // BN,),
        in_specs=[
            pl.BlockSpec((NC, BN, h), lambda i: (0, i, 0)),
            pl.BlockSpec((BN, h), lambda i: (i, 0)),
            pl.BlockSpec((BN, 1), lambda i: (i, 0)),
            pl.BlockSpec((1, h), lambda i: (0, 0)),
            pl.BlockSpec((BN, 1), lambda i: (i, 0)),
        ],
        out_specs=[
            pl.BlockSpec((G, h), lambda i: (0, 0)),
            pl.BlockSpec((G, 1), lambda i: (0, 0)),
        ],
        out_shape=[
            jax.ShapeDtypeStruct((G, h), jnp.float32),
            jax.ShapeDtypeStruct((G, 1), jnp.float32),
        ],
    )(acc, xws, dinv2d, b2d, batch2d)


def _tc_final(sums, cnts, wl, bl2d):
    def body(s_ref, c_ref, w_ref, b_ref, o_ref):
        pooled = s_ref[...] / jnp.maximum(c_ref[...], 1.0)
        o_ref[...] = (
            jnp.dot(pooled, w_ref[...], preferred_element_type=jnp.float32)
            + b_ref[...]
        )

    return pl.pallas_call(
        body,
        out_shape=jax.ShapeDtypeStruct((G, wl.shape[1]), jnp.float32),
    )(sums, cnts, wl, bl2d)


def kernel(x, edge_index, batch, W1, b1, W2, b2, W3, b3, Wl, bl):
    n = x.shape[0]
    e = edge_index.shape[1]
    h = W1.shape[1]

    npad = ((n + 1) + 127) // 128 * 128
    kchunks = (e + NW * CHUNK - 1) // (NW * CHUNK)
    kchunks = (kchunks + SB - 1) // SB * SB
    epad = NW * kchunks * CHUNK

    spare = npad - n
    pad_dst = n + (jnp.arange(epad - e, dtype=jnp.int32) % spare)
    pad_src = jnp.arange(epad - e, dtype=jnp.int32) % n
    src = jnp.concatenate([edge_index[0], pad_src])
    dst = jnp.concatenate([edge_index[1], pad_dst])
    src3 = src.reshape(NW, kchunks, CHUNK)
    dst3 = dst.reshape(NW, kchunks, CHUNK)
    dst2 = dst.reshape(NW, kchunks * CHUNK)
    zrows = jnp.zeros((npad, h), jnp.float32)
    b1r, b2r, b3r = b1.reshape(1, h), b2.reshape(1, h), b3.reshape(1, h)
    blr = bl.reshape(1, bl.shape[0])
    batch2d = batch.reshape(n, 1)

    degp = _sc_degree(dst2, npad)
    dinv2d = _tc_dinv(degp)
    xws1 = _tc_first(x, W1, dinv2d)
    acc1 = _sc_propagate(xws1, src3, dst3, zrows, npad)
    xws2 = _tc_layer(acc1, xws1, dinv2d, b1r, W2)
    acc2 = _sc_propagate(xws2, src3, dst3, zrows, npad)
    xws3 = _tc_layer(acc2, xws2, dinv2d, b2r, W3)
    acc3 = _sc_propagate(xws3, src3, dst3, zrows, npad)
    sums, cnts = _tc_pool(acc3, xws3, dinv2d, b3r, batch2d)
    return _tc_final(sums, cnts, Wl, blr)


# speedup vs baseline: 1.2647x; 1.0663x over previous
import functools

import jax
import jax.numpy as jnp
from jax import lax
from jax.experimental import pallas as pl
from jax.experimental.pallas import tpu as pltpu
from jax.experimental.pallas import tpu_sc as plsc

NC = 2
NS = 16
NW = NC * NS
L = 16
CHUNK = 64
NBUF = 4
SB = 32
GAHEAD = 3
G = 64
BN = 1000


def _mesh():
    return plsc.VectorSubcoreMesh(
        core_axis_name="c", subcore_axis_name="s",
        num_cores=NC, num_subcores=NS)


def _sc_degree(dst2, npad):
    tpt = dst2.shape[1]

    @functools.partial(
        pl.kernel,
        out_type=jax.ShapeDtypeStruct((NW, npad), jnp.float32),
        mesh=_mesh(),
        compiler_params=pltpu.CompilerParams(needs_layout_passes=False),
        scratch_types=[
            pltpu.VMEM((tpt,), jnp.int32),
            pltpu.VMEM((npad,), jnp.float32),
        ],
    )
    def k(dst_hbm, out_hbm, dstv, hist):
        c = lax.axis_index("c")
        s = lax.axis_index("s")
        wid = s * NC + c
        zero16 = jnp.zeros((L,), jnp.float32)

        def zbody(i, carry):
            hist[pl.ds(i * L, L)] = zero16
            return carry

        lax.fori_loop(0, npad // L, zbody, 0)
        pltpu.sync_copy(dst_hbm.at[wid], dstv)
        one16 = jnp.ones((L,), jnp.float32)

        def body(i, carry):
            idx = dstv[pl.ds(i * L, L)]
            plsc.addupdate_scatter(hist, [idx], one16)
            return carry

        lax.fori_loop(0, tpt // L, body, 0)
        pltpu.sync_copy(hist, out_hbm.at[wid])

    return k(dst2)


def _sc_propagate(xws, src3, dst3, zrows, npad):
    kchunks = src3.shape[1]
    nsb = kchunks // SB
    h = xws.shape[1]
    rpt = npad // NS

    @functools.partial(
        pl.kernel,
        out_type=jax.ShapeDtypeStruct((NC, npad, h), jnp.float32),
        mesh=_mesh(),
        compiler_params=pltpu.CompilerParams(needs_layout_passes=False),
        scratch_types=[
            pltpu.VMEM((2, SB, CHUNK), jnp.int32),
            pltpu.VMEM((2, SB, CHUNK), jnp.int32),
            pltpu.VMEM((NBUF, CHUNK, h), jnp.float32),
            pltpu.VMEM_SHARED((npad, h), jnp.float32),
            pltpu.SemaphoreType.DMA,
            pltpu.SemaphoreType.DMA,
            pltpu.SemaphoreType.DMA,
        ],
    )
    def k(xws_hbm, src_hbm, dst_hbm, z_hbm, out_hbm, srcv, dstv, rows, acc,
          gsem, ssem, isem):
        c = lax.axis_index("c")
        s = lax.axis_index("s")
        wid = s * NC + c
        base = s * rpt
        pltpu.sync_copy(z_hbm.at[pl.ds(base, rpt)], acc.at[pl.ds(base, rpt)])
        pltpu.sync_copy(src_hbm.at[wid, pl.ds(0, SB)], srcv.at[0])
        pltpu.sync_copy(dst_hbm.at[wid, pl.ds(0, SB)], dstv.at[0])
        plsc.subcore_barrier()

        def outer(sb, carry):
            slot = sb % 2

            @pl.when(sb + 1 < nsb)
            def _():
                nxt = (sb + 1) % 2
                off = (sb + 1) * SB
                pltpu.async_copy(
                    src_hbm.at[wid, pl.ds(off, SB)], srcv.at[nxt], isem)
                pltpu.async_copy(
                    dst_hbm.at[wid, pl.ds(off, SB)], dstv.at[nxt], isem)

            for p in range(GAHEAD):
                pltpu.async_copy(xws_hbm.at[srcv.at[slot, p]], rows.at[p],
                                 gsem)

            def inner(t, carry2):
                pltpu.make_async_copy(
                    xws_hbm.at[srcv.at[slot, t]], rows.at[t % NBUF],
                    gsem).wait()
                pltpu.async_copy(
                    rows.at[t % NBUF], acc.at[dstv.at[slot, t]], ssem,
                    add=True)

                @pl.when(t >= 1)
                def _():
                    pltpu.make_async_copy(
                        rows.at[(t - 1) % NBUF],
                        acc.at[dstv.at[slot, t - 1]], ssem).wait()

                @pl.when(t + GAHEAD < SB)
                def _():
                    pltpu.async_copy(
                        xws_hbm.at[srcv.at[slot, t + GAHEAD]],
                        rows.at[(t + GAHEAD) % NBUF], gsem)
                return carry2

            lax.fori_loop(0, SB, inner, 0)
            pltpu.make_async_copy(
                rows.at[(SB - 1) % NBUF], acc.at[dstv.at[slot, SB - 1]],
                ssem).wait()

            @pl.when(sb + 1 < nsb)
            def _():
                nxt = (sb + 1) % 2
                off = (sb + 1) * SB
                pltpu.make_async_copy(
                    src_hbm.at[wid, pl.ds(off, SB)], srcv.at[nxt],
                    isem).wait()
                pltpu.make_async_copy(
                    dst_hbm.at[wid, pl.ds(off, SB)], dstv.at[nxt],
                    isem).wait()

            return carry

        lax.fori_loop(0, nsb, outer, 0)
        plsc.subcore_barrier()
        pltpu.sync_copy(acc.at[pl.ds(base, rpt)], out_hbm.at[c, pl.ds(base, rpt)])

    return k(xws, src3, dst3, zrows)


def _tc_dinv(degp):
    nw, npad = degp.shape

    def body(dp_ref, o_ref):
        ones = jnp.ones((nw, 1), jnp.float32)
        deg = 1.0 + lax.dot_general(
            dp_ref[...], ones, (((0,), (0,)), ((), ())),
            preferred_element_type=jnp.float32,
        )
        o_ref[...] = lax.rsqrt(deg)

    return pl.pallas_call(
        body,
        out_shape=jax.ShapeDtypeStruct((npad, 1), jnp.float32),
    )(degp)


def _tc_first(x, w1, dinv2d):
    n, d = x.shape
    h = w1.shape[1]

    def body(x_ref, w_ref, dv_ref, o_ref):
        xw = jnp.dot(x_ref[...], w_ref[...], preferred_element_type=jnp.float32)
        o_ref[...] = xw * dv_ref[...]

    return pl.pallas_call(
        body,
        grid=(n // BN,),
        in_specs=[
            pl.BlockSpec((BN, d), lambda i: (i, 0)),
            pl.BlockSpec((d, h), lambda i: (0, 0)),
            pl.BlockSpec((BN, 1), lambda i: (i, 0)),
        ],
        out_specs=pl.BlockSpec((BN, h), lambda i: (i, 0)),
        out_shape=jax.ShapeDtypeStruct((n, h), jnp.float32),
    )(x, w1, dinv2d)


def _tc_layer(acc, xws, dinv2d, b2d, w):
    n, h = xws.shape
    h2 = w.shape[1]

    def body(a_ref, x_ref, dv_ref, b_ref, w_ref, o_ref):
        dinv = dv_ref[...]
        hpre = (a_ref[0] + a_ref[1] + x_ref[...]) * dinv + b_ref[...]
        hact = jnp.maximum(hpre, 0.0)
        o_ref[...] = (
            jnp.dot(hact, w_ref[...], preferred_element_type=jnp.float32)
            * dinv
        )

    return pl.pallas_call(
        body,
        grid=(n // BN,),
        in_specs=[
            pl.BlockSpec((NC, BN, h), lambda i: (0, i, 0)),
            pl.BlockSpec((BN, h), lambda i: (i, 0)),
            pl.BlockSpec((BN, 1), lambda i: (i, 0)),
            pl.BlockSpec((1, h), lambda i: (0, 0)),
            pl.BlockSpec((h, h2), lambda i: (0, 0)),
        ],
        out_specs=pl.BlockSpec((BN, h2), lambda i: (i, 0)),
        out_shape=jax.ShapeDtypeStruct((n, h2), jnp.float32),
    )(acc, xws, dinv2d, b2d, w)


def _tc_pool(acc, xws, dinv2d, b2d, batch2d):
    n, h = xws.shape

    def body(a_ref, x_ref, dv_ref, b_ref, bt_ref, sums_ref, cnt_ref):
        i = pl.program_id(0)
        hpre = (a_ref[0] + a_ref[1] + x_ref[...]) * dv_ref[...] + b_ref[...]
        hact = jnp.maximum(hpre, 0.0)
        onehot = (
            bt_ref[...] == lax.broadcasted_iota(jnp.int32, (1, G), 1)
        ).astype(jnp.float32)
        psums = lax.dot_general(
            onehot, hact, (((0,), (0,)), ((), ())),
            preferred_element_type=jnp.float32,
        )
        pcnts = lax.dot_general(
            onehot, jnp.ones((BN, 1), jnp.float32), (((0,), (0,)), ((), ())),
            preferred_element_type=jnp.float32,
        )

        @pl.when(i == 0)
        def _():
            sums_ref[...] = jnp.zeros_like(sums_ref)
            cnt_ref[...] = jnp.zeros_like(cnt_ref)

        sums_ref[...] += psums
        cnt_ref[...] += pcnts

    return pl.pallas_call(
        body,
        grid=(n // BN,),
        in_specs=[
            pl.BlockSpec((NC, BN, h), lambda i: (0, i, 0)),
            pl.BlockSpec((BN, h), lambda i: (i, 0)),
            pl.BlockSpec((BN, 1), lambda i: (i, 0)),
            pl.BlockSpec((1, h), lambda i: (0, 0)),
            pl.BlockSpec((BN, 1), lambda i: (i, 0)),
        ],
        out_specs=[
            pl.BlockSpec((G, h), lambda i: (0, 0)),
            pl.BlockSpec((G, 1), lambda i: (0, 0)),
        ],
        out_shape=[
            jax.ShapeDtypeStruct((G, h), jnp.float32),
            jax.ShapeDtypeStruct((G, 1), jnp.float32),
        ],
    )(acc, xws, dinv2d, b2d, batch2d)


def _tc_final(sums, cnts, wl, bl2d):
    def body(s_ref, c_ref, w_ref, b_ref, o_ref):
        pooled = s_ref[...] / jnp.maximum(c_ref[...], 1.0)
        o_ref[...] = (
            jnp.dot(pooled, w_ref[...], preferred_element_type=jnp.float32)
            + b_ref[...]
        )

    return pl.pallas_call(
        body,
        out_shape=jax.ShapeDtypeStruct((G, wl.shape[1]), jnp.float32),
    )(sums, cnts, wl, bl2d)


def kernel(x, edge_index, batch, W1, b1, W2, b2, W3, b3, Wl, bl):
    n = x.shape[0]
    e = edge_index.shape[1]
    h = W1.shape[1]

    npad = ((n + 1) + 127) // 128 * 128
    kchunks = (e + NW * CHUNK - 1) // (NW * CHUNK)
    kchunks = (kchunks + SB - 1) // SB * SB
    epad = NW * kchunks * CHUNK

    spare = npad - n
    pad_dst = n + (jnp.arange(epad - e, dtype=jnp.int32) % spare)
    pad_src = jnp.arange(epad - e, dtype=jnp.int32) % n
    src = jnp.concatenate([edge_index[0], pad_src])
    dst = jnp.concatenate([edge_index[1], pad_dst])
    src3 = src.reshape(NW, kchunks, CHUNK)
    dst3 = dst.reshape(NW, kchunks, CHUNK)
    dst2 = dst.reshape(NW, kchunks * CHUNK)
    zrows = jnp.zeros((npad, h), jnp.float32)
    b1r, b2r, b3r = b1.reshape(1, h), b2.reshape(1, h), b3.reshape(1, h)
    blr = bl.reshape(1, bl.shape[0])
    batch2d = batch.reshape(n, 1)

    degp = _sc_degree(dst2, npad)
    dinv2d = _tc_dinv(degp)
    xws1 = _tc_first(x, W1, dinv2d)
    acc1 = _sc_propagate(xws1, src3, dst3, zrows, npad)
    xws2 = _tc_layer(acc1, xws1, dinv2d, b1r, W2)
    acc2 = _sc_propagate(xws2, src3, dst3, zrows, npad)
    xws3 = _tc_layer(acc2, xws2, dinv2d, b2r, W3)
    acc3 = _sc_propagate(xws3, src3, dst3, zrows, npad)
    sums, cnts = _tc_pool(acc3, xws3, dinv2d, b3r, batch2d)
    return _tc_final(sums, cnts, Wl, blr)


# speedup vs baseline: 1.2915x; 1.0212x over previous
import functools

import jax
import jax.numpy as jnp
from jax import lax
from jax.experimental import pallas as pl
from jax.experimental.pallas import tpu as pltpu
from jax.experimental.pallas import tpu_sc as plsc

NC = 2
NS = 16
NW = NC * NS
L = 16
CHUNK = 64
NBUF = 4
SB = 32
GAHEAD = 3
G = 64
BN = 1000


def _mesh():
    return plsc.VectorSubcoreMesh(
        core_axis_name="c", subcore_axis_name="s",
        num_cores=NC, num_subcores=NS)


def _sc_degree(dst2, npad):
    tpt = dst2.shape[1]

    @functools.partial(
        pl.kernel,
        out_type=jax.ShapeDtypeStruct((NW, npad), jnp.float32),
        mesh=_mesh(),
        compiler_params=pltpu.CompilerParams(needs_layout_passes=False),
        scratch_types=[
            pltpu.VMEM((tpt,), jnp.int32),
            pltpu.VMEM((npad,), jnp.float32),
        ],
    )
    def k(dst_hbm, out_hbm, dstv, hist):
        c = lax.axis_index("c")
        s = lax.axis_index("s")
        wid = s * NC + c
        zero16 = jnp.zeros((L,), jnp.float32)

        def zbody(i, carry):
            hist[pl.ds(i * L, L)] = zero16
            return carry

        lax.fori_loop(0, npad // L, zbody, 0)
        pltpu.sync_copy(dst_hbm.at[wid], dstv)
        one16 = jnp.ones((L,), jnp.float32)

        def body(i, carry):
            idx = dstv[pl.ds(i * L, L)]
            plsc.addupdate_scatter(hist, [idx], one16)
            return carry

        lax.fori_loop(0, tpt // L, body, 0)
        pltpu.sync_copy(hist, out_hbm.at[wid])

    return k(dst2)


def _sc_propagate(xws, src3, dst3, zrows, npad):
    kchunks = src3.shape[1]
    nsb = kchunks // SB
    h = xws.shape[1]
    rpt = npad // NS

    @functools.partial(
        pl.kernel,
        out_type=jax.ShapeDtypeStruct((NC, npad, h), jnp.float32),
        mesh=_mesh(),
        compiler_params=pltpu.CompilerParams(needs_layout_passes=False),
        scratch_types=[
            pltpu.VMEM((2, SB, CHUNK), jnp.int32),
            pltpu.VMEM((2, SB, CHUNK), jnp.int32),
            pltpu.VMEM((NBUF, CHUNK, h), jnp.float32),
            pltpu.VMEM_SHARED((npad, h), jnp.float32),
            pltpu.SemaphoreType.DMA,
            pltpu.SemaphoreType.DMA,
            pltpu.SemaphoreType.DMA,
        ],
    )
    def k(xws_hbm, src_hbm, dst_hbm, z_hbm, out_hbm, srcv, dstv, rows, acc,
          gsem, ssem, isem):
        c = lax.axis_index("c")
        s = lax.axis_index("s")
        wid = s * NC + c
        base = s * rpt
        zdesc = pltpu.async_copy(
            z_hbm.at[pl.ds(base, rpt)], acc.at[pl.ds(base, rpt)], ssem)
        pltpu.sync_copy(src_hbm.at[wid, pl.ds(0, SB)], srcv.at[0])
        pltpu.sync_copy(dst_hbm.at[wid, pl.ds(0, SB)], dstv.at[0])

        def outer(sb, carry):
            slot = sb % 2

            @pl.when(sb + 1 < nsb)
            def _():
                nxt = (sb + 1) % 2
                off = (sb + 1) * SB
                pltpu.async_copy(
                    src_hbm.at[wid, pl.ds(off, SB)], srcv.at[nxt], isem)
                pltpu.async_copy(
                    dst_hbm.at[wid, pl.ds(off, SB)], dstv.at[nxt], isem)

            for p in range(GAHEAD):
                pltpu.async_copy(xws_hbm.at[srcv.at[slot, p]], rows.at[p],
                                 gsem)

            @pl.when(sb == 0)
            def _():
                pltpu.make_async_copy(
                    z_hbm.at[pl.ds(base, rpt)], acc.at[pl.ds(base, rpt)],
                    ssem).wait()
                plsc.subcore_barrier()

            def inner(t, carry2):
                pltpu.make_async_copy(
                    xws_hbm.at[srcv.at[slot, t]], rows.at[t % NBUF],
                    gsem).wait()
                pltpu.async_copy(
                    rows.at[t % NBUF], acc.at[dstv.at[slot, t]], ssem,
                    add=True)

                @pl.when(t >= 1)
                def _():
                    pltpu.make_async_copy(
                        rows.at[(t - 1) % NBUF],
                        acc.at[dstv.at[slot, t - 1]], ssem).wait()

                @pl.when(t + GAHEAD < SB)
                def _():
                    pltpu.async_copy(
                        xws_hbm.at[srcv.at[slot, t + GAHEAD]],
                        rows.at[(t + GAHEAD) % NBUF], gsem)
                return carry2

            lax.fori_loop(0, SB, inner, 0)
            pltpu.make_async_copy(
                rows.at[(SB - 1) % NBUF], acc.at[dstv.at[slot, SB - 1]],
                ssem).wait()

            @pl.when(sb + 1 < nsb)
            def _():
                nxt = (sb + 1) % 2
                off = (sb + 1) * SB
                pltpu.make_async_copy(
                    src_hbm.at[wid, pl.ds(off, SB)], srcv.at[nxt],
                    isem).wait()
                pltpu.make_async_copy(
                    dst_hbm.at[wid, pl.ds(off, SB)], dstv.at[nxt],
                    isem).wait()

            return carry

        lax.fori_loop(0, nsb, outer, 0)
        plsc.subcore_barrier()
        pltpu.sync_copy(acc.at[pl.ds(base, rpt)], out_hbm.at[c, pl.ds(base, rpt)])

    return k(xws, src3, dst3, zrows)


def _tc_dinv(degp):
    nw, npad = degp.shape

    def body(dp_ref, o_ref):
        ones = jnp.ones((nw, 1), jnp.float32)
        deg = 1.0 + lax.dot_general(
            dp_ref[...], ones, (((0,), (0,)), ((), ())),
            preferred_element_type=jnp.float32,
        )
        o_ref[...] = lax.rsqrt(deg)

    return pl.pallas_call(
        body,
        out_shape=jax.ShapeDtypeStruct((npad, 1), jnp.float32),
    )(degp)


def _tc_first(x, w1, dinv2d):
    n, d = x.shape
    h = w1.shape[1]

    def body(x_ref, w_ref, dv_ref, o_ref):
        xw = jnp.dot(x_ref[...], w_ref[...], preferred_element_type=jnp.float32)
        o_ref[...] = xw * dv_ref[...]

    return pl.pallas_call(
        body,
        grid=(n // BN,),
        in_specs=[
            pl.BlockSpec((BN, d), lambda i: (i, 0)),
            pl.BlockSpec((d, h), lambda i: (0, 0)),
            pl.BlockSpec((BN, 1), lambda i: (i, 0)),
        ],
        out_specs=pl.BlockSpec((BN, h), lambda i: (i, 0)),
        out_shape=jax.ShapeDtypeStruct((n, h), jnp.float32),
    )(x, w1, dinv2d)


def _tc_layer(acc, xws, dinv2d, b2d, w):
    n, h = xws.shape
    h2 = w.shape[1]

    def body(a_ref, x_ref, dv_ref, b_ref, w_ref, o_ref):
        dinv = dv_ref[...]
        hpre = (a_ref[0] + a_ref[1] + x_ref[...]) * dinv + b_ref[...]
        hact = jnp.maximum(hpre, 0.0)
        o_ref[...] = (
            jnp.dot(hact, w_ref[...], preferred_element_type=jnp.float32)
            * dinv
        )

    return pl.pallas_call(
        body,
        grid=(n // BN,),
        in_specs=[
            pl.BlockSpec((NC, BN, h), lambda i: (0, i, 0)),
            pl.BlockSpec((BN, h), lambda i: (i, 0)),
            pl.BlockSpec((BN, 1), lambda i: (i, 0)),
            pl.BlockSpec((1, h), lambda i: (0, 0)),
            pl.BlockSpec((h, h2), lambda i: (0, 0)),
        ],
        out_specs=pl.BlockSpec((BN, h2), lambda i: (i, 0)),
        out_shape=jax.ShapeDtypeStruct((n, h2), jnp.float32),
    )(acc, xws, dinv2d, b2d, w)


def _tc_pool(acc, xws, dinv2d, b2d, batch2d):
    n, h = xws.shape

    def body(a_ref, x_ref, dv_ref, b_ref, bt_ref, sums_ref, cnt_ref):
        i = pl.program_id(0)
        hpre = (a_ref[0] + a_ref[1] + x_ref[...]) * dv_ref[...] + b_ref[...]
        hact = jnp.maximum(hpre, 0.0)
        onehot = (
            bt_ref[...] == lax.broadcasted_iota(jnp.int32, (1, G), 1)
        ).astype(jnp.float32)
        psums = lax.dot_general(
            onehot, hact, (((0,), (0,)), ((), ())),
            preferred_element_type=jnp.float32,
        )
        pcnts = lax.dot_general(
            onehot, jnp.ones((BN, 1), jnp.float32), (((0,), (0,)), ((), ())),
            preferred_element_type=jnp.float32,
        )

        @pl.when(i == 0)
        def _():
            sums_ref[...] = jnp.zeros_like(sums_ref)
            cnt_ref[...] = jnp.zeros_like(cnt_ref)

        sums_ref[...] += psums
        cnt_ref[...] += pcnts

    return pl.pallas_call(
        body,
        grid=(n // BN,),
        in_specs=[
            pl.BlockSpec((NC, BN, h), lambda i: (0, i, 0)),
            pl.BlockSpec((BN, h), lambda i: (i, 0)),
            pl.BlockSpec((BN, 1), lambda i: (i, 0)),
            pl.BlockSpec((1, h), lambda i: (0, 0)),
            pl.BlockSpec((BN, 1), lambda i: (i, 0)),
        ],
        out_specs=[
            pl.BlockSpec((G, h), lambda i: (0, 0)),
            pl.BlockSpec((G, 1), lambda i: (0, 0)),
        ],
        out_shape=[
            jax.ShapeDtypeStruct((G, h), jnp.float32),
            jax.ShapeDtypeStruct((G, 1), jnp.float32),
        ],
    )(acc, xws, dinv2d, b2d, batch2d)


def _tc_final(sums, cnts, wl, bl2d):
    def body(s_ref, c_ref, w_ref, b_ref, o_ref):
        pooled = s_ref[...] / jnp.maximum(c_ref[...], 1.0)
        o_ref[...] = (
            jnp.dot(pooled, w_ref[...], preferred_element_type=jnp.float32)
            + b_ref[...]
        )

    return pl.pallas_call(
        body,
        out_shape=jax.ShapeDtypeStruct((G, wl.shape[1]), jnp.float32),
    )(sums, cnts, wl, bl2d)


def kernel(x, edge_index, batch, W1, b1, W2, b2, W3, b3, Wl, bl):
    n = x.shape[0]
    e = edge_index.shape[1]
    h = W1.shape[1]

    npad = ((n + 1) + 127) // 128 * 128
    kchunks = (e + NW * CHUNK - 1) // (NW * CHUNK)
    kchunks = (kchunks + SB - 1) // SB * SB
    epad = NW * kchunks * CHUNK

    spare = npad - n
    pad_dst = n + (jnp.arange(epad - e, dtype=jnp.int32) % spare)
    pad_src = jnp.arange(epad - e, dtype=jnp.int32) % n
    src = jnp.concatenate([edge_index[0], pad_src])
    dst = jnp.concatenate([edge_index[1], pad_dst])
    src3 = src.reshape(NW, kchunks, CHUNK)
    dst3 = dst.reshape(NW, kchunks, CHUNK)
    dst2 = dst.reshape(NW, kchunks * CHUNK)
    zrows = jnp.zeros((npad, h), jnp.float32)
    b1r, b2r, b3r = b1.reshape(1, h), b2.reshape(1, h), b3.reshape(1, h)
    blr = bl.reshape(1, bl.shape[0])
    batch2d = batch.reshape(n, 1)

    degp = _sc_degree(dst2, npad)
    dinv2d = _tc_dinv(degp)
    xws1 = _tc_first(x, W1, dinv2d)
    acc1 = _sc_propagate(xws1, src3, dst3, zrows, npad)
    xws2 = _tc_layer(acc1, xws1, dinv2d, b1r, W2)
    acc2 = _sc_propagate(xws2, src3, dst3, zrows, npad)
    xws3 = _tc_layer(acc2, xws2, dinv2d, b2r, W3)
    acc3 = _sc_propagate(xws3, src3, dst3, zrows, npad)
    sums, cnts = _tc_pool(acc3, xws3, dinv2d, b3r, batch2d)
    return _tc_final(sums, cnts, Wl, blr)


# speedup vs baseline: 1.3206x; 1.0226x over previous
import functools

import jax
import jax.numpy as jnp
from jax import lax
from jax.experimental import pallas as pl
from jax.experimental.pallas import tpu as pltpu
from jax.experimental.pallas import tpu_sc as plsc

NC = 2
NS = 16
NW = NC * NS
L = 16
CHUNK = 64
NBUF = 4
SB = 32
GAHEAD = 3
G = 64
BN = 1000


def _mesh():
    return plsc.VectorSubcoreMesh(
        core_axis_name="c", subcore_axis_name="s",
        num_cores=NC, num_subcores=NS)


def _sc_degree(dst2, npad):
    tpt = dst2.shape[1]

    @functools.partial(
        pl.kernel,
        out_type=jax.ShapeDtypeStruct((NW, npad), jnp.float32),
        mesh=_mesh(),
        compiler_params=pltpu.CompilerParams(needs_layout_passes=False),
        scratch_types=[
            pltpu.VMEM((tpt,), jnp.int32),
            pltpu.VMEM((npad,), jnp.float32),
        ],
    )
    def k(dst_hbm, out_hbm, dstv, hist):
        c = lax.axis_index("c")
        s = lax.axis_index("s")
        wid = s * NC + c
        zero16 = jnp.zeros((L,), jnp.float32)

        def zbody(i, carry):
            hist[pl.ds(i * L, L)] = zero16
            return carry

        lax.fori_loop(0, npad // L, zbody, 0)
        pltpu.sync_copy(dst_hbm.at[wid], dstv)
        one16 = jnp.ones((L,), jnp.float32)

        def body(i, carry):
            idx = dstv[pl.ds(i * L, L)]
            plsc.addupdate_scatter(hist, [idx], one16)
            return carry

        lax.fori_loop(0, tpt // L, body, 0)
        pltpu.sync_copy(hist, out_hbm.at[wid])

    return k(dst2)


def _sc_propagate(xws, src3, dst3, zrows, npad):
    kchunks = src3.shape[1]
    nsb = kchunks // SB
    h = xws.shape[1]
    rpt = npad // NS

    @functools.partial(
        pl.kernel,
        out_type=jax.ShapeDtypeStruct((NC, npad, h), jnp.float32),
        mesh=_mesh(),
        compiler_params=pltpu.CompilerParams(needs_layout_passes=False),
        scratch_types=[
            pltpu.VMEM((2, SB, CHUNK), jnp.int32),
            pltpu.VMEM((2, SB, CHUNK), jnp.int32),
            pltpu.VMEM((NBUF, CHUNK, h), jnp.float32),
            pltpu.VMEM_SHARED((npad, h), jnp.float32),
            pltpu.SemaphoreType.DMA,
            pltpu.SemaphoreType.DMA,
            pltpu.SemaphoreType.DMA,
        ],
    )
    def k(xws_hbm, src_hbm, dst_hbm, z_hbm, out_hbm, srcv, dstv, rows, acc,
          gsem, ssem, isem):
        c = lax.axis_index("c")
        s = lax.axis_index("s")
        wid = s * NC + c
        base = s * rpt
        zdesc = pltpu.async_copy(
            z_hbm.at[pl.ds(base, rpt)], acc.at[pl.ds(base, rpt)], ssem)
        pltpu.sync_copy(src_hbm.at[wid, pl.ds(0, SB)], srcv.at[0])
        pltpu.sync_copy(dst_hbm.at[wid, pl.ds(0, SB)], dstv.at[0])

        def outer(sb, carry):
            slot = sb % 2

            @pl.when(sb + 1 < nsb)
            def _():
                nxt = (sb + 1) % 2
                off = (sb + 1) * SB
                pltpu.async_copy(
                    src_hbm.at[wid, pl.ds(off, SB)], srcv.at[nxt], isem)
                pltpu.async_copy(
                    dst_hbm.at[wid, pl.ds(off, SB)], dstv.at[nxt], isem)

            for p in range(GAHEAD):
                pltpu.async_copy(xws_hbm.at[srcv.at[slot, p]], rows.at[p],
                                 gsem)

            @pl.when(sb == 0)
            def _():
                pltpu.make_async_copy(
                    z_hbm.at[pl.ds(base, rpt)], acc.at[pl.ds(base, rpt)],
                    ssem).wait()
                plsc.subcore_barrier()

            def inner(t, carry2):
                pltpu.make_async_copy(
                    xws_hbm.at[srcv.at[slot, t]], rows.at[t % NBUF],
                    gsem).wait()
                pltpu.async_copy(
                    rows.at[t % NBUF], acc.at[dstv.at[slot, t]], ssem,
                    add=True)

                @pl.when(t >= 1)
                def _():
                    pltpu.make_async_copy(
                        rows.at[(t - 1) % NBUF],
                        acc.at[dstv.at[slot, t - 1]], ssem).wait()

                @pl.when(t + GAHEAD < SB)
                def _():
                    pltpu.async_copy(
                        xws_hbm.at[srcv.at[slot, t + GAHEAD]],
                        rows.at[(t + GAHEAD) % NBUF], gsem)
                return carry2

            lax.fori_loop(0, SB, inner, 0)
            pltpu.make_async_copy(
                rows.at[(SB - 1) % NBUF], acc.at[dstv.at[slot, SB - 1]],
                ssem).wait()

            @pl.when(sb + 1 < nsb)
            def _():
                nxt = (sb + 1) % 2
                off = (sb + 1) * SB
                pltpu.make_async_copy(
                    src_hbm.at[wid, pl.ds(off, SB)], srcv.at[nxt],
                    isem).wait()
                pltpu.make_async_copy(
                    dst_hbm.at[wid, pl.ds(off, SB)], dstv.at[nxt],
                    isem).wait()

            return carry

        lax.fori_loop(0, nsb, outer, 0)
        plsc.subcore_barrier()
        pltpu.sync_copy(acc.at[pl.ds(base, rpt)], out_hbm.at[c, pl.ds(base, rpt)])

    return k(xws, src3, dst3, zrows)


def _tc_first(x, w1, degp):
    n, d = x.shape
    h = w1.shape[1]
    nw = degp.shape[0]

    def body(x_ref, w_ref, dp_ref, o_ref, dv_ref):
        ones = jnp.ones((nw, 1), jnp.float32)
        deg = 1.0 + lax.dot_general(
            dp_ref[...], ones, (((0,), (0,)), ((), ())),
            preferred_element_type=jnp.float32,
        )
        dinv = lax.rsqrt(deg)[:n]
        dv_ref[...] = dinv
        xw = jnp.dot(x_ref[...], w_ref[...], preferred_element_type=jnp.float32)
        o_ref[...] = xw * dinv

    return pl.pallas_call(
        body,
        out_shape=[
            jax.ShapeDtypeStruct((n, h), jnp.float32),
            jax.ShapeDtypeStruct((n, 1), jnp.float32),
        ],
    )(x, w1, degp)


def _tc_layer(acc, xws, dinv2d, b2d, w):
    n, h = xws.shape
    h2 = w.shape[1]

    def body(a_ref, x_ref, dv_ref, b_ref, w_ref, o_ref):
        dinv = dv_ref[...]
        hpre = (a_ref[0] + a_ref[1] + x_ref[...]) * dinv + b_ref[...]
        hact = jnp.maximum(hpre, 0.0)
        o_ref[...] = (
            jnp.dot(hact, w_ref[...], preferred_element_type=jnp.float32)
            * dinv
        )

    return pl.pallas_call(
        body,
        grid=(n // BN,),
        in_specs=[
            pl.BlockSpec((NC, BN, h), lambda i: (0, i, 0)),
            pl.BlockSpec((BN, h), lambda i: (i, 0)),
            pl.BlockSpec((BN, 1), lambda i: (i, 0)),
            pl.BlockSpec((1, h), lambda i: (0, 0)),
            pl.BlockSpec((h, h2), lambda i: (0, 0)),
        ],
        out_specs=pl.BlockSpec((BN, h2), lambda i: (i, 0)),
        out_shape=jax.ShapeDtypeStruct((n, h2), jnp.float32),
    )(acc, xws, dinv2d, b2d, w)


def _tc_pool(acc, xws, dinv2d, b2d, batch2d, wl, bl2d):
    n, h = xws.shape
    c = wl.shape[1]
    ngrid = n // BN

    def body(a_ref, x_ref, dv_ref, b_ref, bt_ref, wl_ref, bl_ref, o_ref,
             sums_ref, cnt_ref):
        i = pl.program_id(0)
        hpre = (a_ref[0] + a_ref[1] + x_ref[...]) * dv_ref[...] + b_ref[...]
        hact = jnp.maximum(hpre, 0.0)
        onehot = (
            bt_ref[...] == lax.broadcasted_iota(jnp.int32, (1, G), 1)
        ).astype(jnp.float32)
        psums = lax.dot_general(
            onehot, hact, (((0,), (0,)), ((), ())),
            preferred_element_type=jnp.float32,
        )
        pcnts = lax.dot_general(
            onehot, jnp.ones((BN, 1), jnp.float32), (((0,), (0,)), ((), ())),
            preferred_element_type=jnp.float32,
        )

        @pl.when(i == 0)
        def _():
            sums_ref[...] = jnp.zeros_like(sums_ref)
            cnt_ref[...] = jnp.zeros_like(cnt_ref)

        sums_ref[...] += psums
        cnt_ref[...] += pcnts

        @pl.when(i == ngrid - 1)
        def _():
            pooled = sums_ref[...] / jnp.maximum(cnt_ref[...], 1.0)
            o_ref[...] = (
                jnp.dot(pooled, wl_ref[...],
                        preferred_element_type=jnp.float32)
                + bl_ref[...]
            )

    return pl.pallas_call(
        body,
        grid=(ngrid,),
        in_specs=[
            pl.BlockSpec((NC, BN, h), lambda i: (0, i, 0)),
            pl.BlockSpec((BN, h), lambda i: (i, 0)),
            pl.BlockSpec((BN, 1), lambda i: (i, 0)),
            pl.BlockSpec((1, h), lambda i: (0, 0)),
            pl.BlockSpec((BN, 1), lambda i: (i, 0)),
            pl.BlockSpec((h, c), lambda i: (0, 0)),
            pl.BlockSpec((1, c), lambda i: (0, 0)),
        ],
        out_specs=pl.BlockSpec((G, c), lambda i: (0, 0)),
        out_shape=jax.ShapeDtypeStruct((G, c), jnp.float32),
        scratch_shapes=[
            pltpu.VMEM((G, h), jnp.float32),
            pltpu.VMEM((G, 1), jnp.float32),
        ],
    )(acc, xws, dinv2d, b2d, batch2d, wl, bl2d)


def kernel(x, edge_index, batch, W1, b1, W2, b2, W3, b3, Wl, bl):
    n = x.shape[0]
    e = edge_index.shape[1]
    h = W1.shape[1]

    npad = ((n + 1) + 127) // 128 * 128
    kchunks = (e + NW * CHUNK - 1) // (NW * CHUNK)
    kchunks = (kchunks + SB - 1) // SB * SB
    epad = NW * kchunks * CHUNK

    spare = npad - n
    pad_dst = n + (jnp.arange(epad - e, dtype=jnp.int32) % spare)
    pad_src = jnp.arange(epad - e, dtype=jnp.int32) % n
    src = jnp.concatenate([edge_index[0], pad_src])
    dst = jnp.concatenate([edge_index[1], pad_dst])
    src3 = src.reshape(NW, kchunks, CHUNK)
    dst3 = dst.reshape(NW, kchunks, CHUNK)
    dst2 = dst.reshape(NW, kchunks * CHUNK)
    zrows = jnp.zeros((npad, h), jnp.float32)
    b1r, b2r, b3r = b1.reshape(1, h), b2.reshape(1, h), b3.reshape(1, h)
    blr = bl.reshape(1, bl.shape[0])
    batch2d = batch.reshape(n, 1)

    degp = _sc_degree(dst2, npad)
    xws1, dinv2d = _tc_first(x, W1, degp)
    acc1 = _sc_propagate(xws1, src3, dst3, zrows, npad)
    xws2 = _tc_layer(acc1, xws1, dinv2d, b1r, W2)
    acc2 = _sc_propagate(xws2, src3, dst3, zrows, npad)
    xws3 = _tc_layer(acc2, xws2, dinv2d, b2r, W3)
    acc3 = _sc_propagate(xws3, src3, dst3, zrows, npad)
    return _tc_pool(acc3, xws3, dinv2d, b3r, batch2d, Wl, blr)
